# Initial kernel scaffold; baseline (speedup 1.0000x reference)
import jax
import jax.numpy as jnp
from jax.experimental import pallas as pl


def kernel(node_features, edge_features, W_enc1, b_enc1, W_enc2, b_enc2, conv_W1, conv_b1, conv_W2, conv_b2, skip_W, skip_b, ln_g, ln_b, poly_W, poly_b, edge_index, n):
    raise NotImplementedError("write your pallas kernel here")


# speedup vs baseline: 2.5987x; 2.5987x over previous
import functools

import jax
import jax.numpy as jnp
from jax import lax
from jax.experimental import pallas as pl
from jax.experimental.pallas import tpu as pltpu
from jax.experimental.pallas import tpu_sc as plsc

N_LAYERS = 3
EMB = 128
EDIM = 16
N_NODES = 10000
N_EDGES = 320000

NC = 2
NS = 16
NW = NC * NS
EDGES_PER_W = N_EDGES // NW
CHUNK = 80
N_CHUNKS = EDGES_PER_W // CHUNK
AGGW = EMB


def _gelu_exact(x):
    return 0.5 * x * (1.0 + lax.erf(x * 0.7071067811865476))


def _gelu_tanh16(x):
    z = 0.7978845608028654 * (x + 0.044715 * x * x * x)
    t = 1.0 - 2.0 / (jnp.exp(2.0 * z) + 1.0)
    return 0.5 * x * (1.0 + t)


def _enc_body(nf, w1, b1, w2, b2, out):
    h = _gelu_exact(jnp.dot(nf[...], w1[...], preferred_element_type=jnp.float32) + b1[...])
    out[...] = jnp.dot(h, w2[...], preferred_element_type=jnp.float32) + b2[...]


def _prep_body(h, w1s, w1d, b1, sw, sb, a_out, b_out, s_out):
    hv = h[...]
    a_out[...] = jnp.dot(hv, w1s[...], preferred_element_type=jnp.float32) + b1[...]
    b_out[...] = jnp.dot(hv, w1d[...], preferred_element_type=jnp.float32)
    s_out[...] = jnp.dot(hv, sw[...], preferred_element_type=jnp.float32) + sb[...]


def _emat_body(ef, we, e_out):
    e_out[...] = jnp.dot(ef[...], we[...], preferred_element_type=jnp.float32)


def _post_body(u2, w2, s, g, bb, out):
    u = u2[0] + u2[1]
    agg = jnp.dot(u, w2[...], preferred_element_type=jnp.float32)
    v = agg + s[...]
    mu = jnp.mean(v, axis=-1, keepdims=True)
    d = v - mu
    var = jnp.mean(d * d, axis=-1, keepdims=True)
    y = d * jax.lax.rsqrt(var + 1e-5) * g[...] + bb[...]
    out[...] = _gelu_exact(y)


def _head_body(h, pw, pb, out):
    out[...] = jnp.dot(h[...], pw[...], preferred_element_type=jnp.float32) + pb[...]


def _full(shape):
    return pl.BlockSpec(shape, lambda i: tuple(0 for _ in shape))


def _enc(nf, w1, b1, w2, b2):
    blk, grid = 1000, N_NODES // 1000
    return pl.pallas_call(
        _enc_body,
        grid=(grid,),
        in_specs=[pl.BlockSpec((blk, EMB), lambda i: (i, 0)),
                  _full((EMB, EMB)), _full((1, EMB)), _full((EMB, EMB)), _full((1, EMB))],
        out_specs=pl.BlockSpec((blk, EMB), lambda i: (i, 0)),
        out_shape=jax.ShapeDtypeStruct((N_NODES, EMB), jnp.float32),
    )(nf, w1, b1, w2, b2)


def _prep(h, w1s, w1d, b1, sw, sb):
    blk, grid = 1000, N_NODES // 1000
    o = jax.ShapeDtypeStruct((N_NODES, EMB), jnp.float32)
    return pl.pallas_call(
        _prep_body,
        grid=(grid,),
        in_specs=[pl.BlockSpec((blk, EMB), lambda i: (i, 0)),
                  _full((EMB, EMB)), _full((EMB, EMB)), _full((1, EMB)),
                  _full((EMB, EMB)), _full((1, EMB))],
        out_specs=[pl.BlockSpec((blk, EMB), lambda i: (i, 0))] * 3,
        out_shape=[o, o, o],
    )(h, w1s, w1d, b1, sw, sb)


def _emat(ef, we):
    blk = 2000
    grid = N_EDGES // blk
    return pl.pallas_call(
        _emat_body,
        grid=(grid,),
        in_specs=[pl.BlockSpec((blk, EDIM), lambda i: (i, 0)), _full((EDIM, EMB))],
        out_specs=pl.BlockSpec((blk, EMB), lambda i: (i, 0)),
        out_shape=jax.ShapeDtypeStruct((N_EDGES, EMB), jnp.float32),
    )(ef, we)


def _post(u2, w2, s, g, bb):
    blk, grid = 1000, N_NODES // 1000
    return pl.pallas_call(
        _post_body,
        grid=(grid,),
        in_specs=[pl.BlockSpec((NC, blk, AGGW), lambda i: (0, i, 0)),
                  _full((EMB, EMB)),
                  pl.BlockSpec((blk, EMB), lambda i: (i, 0)),
                  _full((1, EMB)), _full((1, EMB))],
        out_specs=pl.BlockSpec((blk, EMB), lambda i: (i, 0)),
        out_shape=jax.ShapeDtypeStruct((N_NODES, EMB), jnp.float32),
    )(u2, w2, s, g, bb)


def _head(h, pw, pb):
    blk, grid = 1000, N_NODES // 1000
    k = pw.shape[-1]
    return pl.pallas_call(
        _head_body,
        grid=(grid,),
        in_specs=[pl.BlockSpec((blk, EMB), lambda i: (i, 0)),
                  _full((EMB, k)), _full((1, k))],
        out_specs=pl.BlockSpec((blk, k), lambda i: (i, 0)),
        out_shape=jax.ShapeDtypeStruct((N_NODES, k), jnp.float32),
    )(h, pw, pb)


def _sc_edge_body(a_hbm, b_hbm, e_hbm, rows_hbm, cols_hbm, zero_hbm, c_hbm,
                  out_hbm, idx_r, idx_c, buf_a, buf_b, buf_e, buf_g, buf_c,
                  agg_sh, sem):
    cid = lax.axis_index("c")
    sid = lax.axis_index("s")
    wid = sid * NC + cid

    @pl.when(sid == 0)
    def _():
        pltpu.sync_copy(zero_hbm, agg_sh)

    pltpu.sync_copy(c_hbm, buf_c)
    plsc.subcore_barrier()

    def chunk_body(t, _):
        base = wid * EDGES_PER_W + t * CHUNK
        pltpu.sync_copy(rows_hbm.at[pl.ds(base, CHUNK)], idx_r)
        pltpu.sync_copy(cols_hbm.at[pl.ds(base, CHUNK)], idx_c)
        pltpu.sync_copy(e_hbm.at[pl.ds(base, CHUNK), :], buf_e)
        cp_a = pltpu.async_copy(a_hbm.at[idx_r], buf_a, sem)
        cp_b = pltpu.async_copy(b_hbm.at[idx_c], buf_b, sem)
        cp_a.wait()
        cp_b.wait()

        def edge_body(e, _):
            for c8 in range(EMB // 16):
                sl = pl.ds(c8 * 16, 16)
                x = buf_a[e, sl] + buf_b[e, sl] + buf_e[e, sl]
                buf_g[e, sl] = _gelu_tanh16(x) + buf_c[sl]
            return 0

        lax.fori_loop(0, CHUNK, edge_body, 0)
        pltpu.sync_copy(buf_g, agg_sh.at[idx_r], add=True)
        return 0

    lax.fori_loop(0, N_CHUNKS, chunk_body, 0)
    plsc.subcore_barrier()

    @pl.when(sid == 0)
    def _():
        pltpu.sync_copy(agg_sh, out_hbm.at[cid])


@functools.lru_cache(maxsize=1)
def _sc_edge_fn():
    return pl.kernel(
        _sc_edge_body,
        out_type=jax.ShapeDtypeStruct((NC, N_NODES, AGGW), jnp.float32),
        mesh=plsc.VectorSubcoreMesh(core_axis_name="c", subcore_axis_name="s"),
        scratch_types=[
            pltpu.VMEM((CHUNK,), jnp.int32),
            pltpu.VMEM((CHUNK,), jnp.int32),
            pltpu.VMEM((CHUNK, EMB), jnp.float32),
            pltpu.VMEM((CHUNK, EMB), jnp.float32),
            pltpu.VMEM((CHUNK, EMB), jnp.float32),
            pltpu.VMEM((CHUNK, AGGW), jnp.float32),
            pltpu.VMEM((EMB,), jnp.float32),
            pltpu.VMEM_SHARED((N_NODES, AGGW), jnp.float32),
            pltpu.SemaphoreType.DMA,
        ],
    )


def kernel(node_features, edge_features, W_enc1, b_enc1, W_enc2, b_enc2,
           conv_W1, conv_b1, conv_W2, conv_b2, skip_W, skip_b, ln_g, ln_b,
           poly_W, poly_b, edge_index, n):
    r2 = lambda v: v.reshape(1, -1)
    rows = edge_index[0]
    cols = edge_index[1]
    zero = jnp.zeros((N_NODES, AGGW), jnp.float32)

    h = _enc(node_features, W_enc1, r2(b_enc1), W_enc2, r2(b_enc2))
    for i in range(N_LAYERS):
        w1 = conv_W1[i]
        c_vec = jnp.linalg.solve(conv_W2[i].T, conv_b2[i])
        a_t, b_t, s_t = _prep(h, w1[:EMB], w1[EMB:2 * EMB], r2(conv_b1[i]),
                              skip_W[i], r2(skip_b[i]))
        e_t = _emat(edge_features, w1[2 * EMB:])
        u2 = _sc_edge_fn()(a_t, b_t, e_t, rows, cols, zero, c_vec)
        h = _post(u2, conv_W2[i], s_t, r2(ln_g[i]), r2(ln_b[i]))
    return _head(h, poly_W, r2(poly_b))
